# baseline (speedup 1.0000x reference)
import functools

import jax
import jax.numpy as jnp
from jax import lax
from jax.experimental import pallas as pl
from jax.experimental.pallas import tpu as pltpu
from jax.experimental.pallas import tpu_sc as plsc

_NC = 2
_NS = 16
_NW = _NC * _NS
_LANES = 16


@functools.partial(jax.jit, static_argnames=("interpret",))
def _run(x_blk, table, pos_enc, interpret=False):
    NW, L, BPW = x_blk.shape
    V, D = table.shape
    nvec = D // _LANES
    scale = float(D) ** 0.5

    mesh = plsc.VectorSubcoreMesh(
        core_axis_name="c", subcore_axis_name="s",
        num_cores=_NC, num_subcores=_NS,
    )

    def body(xr, tab, pos, out, idx_v, pos_v, in0, in1, o0, o1, sem_g, sem_o):
        c = lax.axis_index("c")
        s = lax.axis_index("s")
        w = s * _NC + c
        base = w * BPW

        pltpu.sync_copy(xr.at[w], idx_v)
        pltpu.sync_copy(pos, pos_v)

        ins = (in0, in1)
        outs = (o0, o1)

        pltpu.async_copy(tab.at[idx_v.at[0]], in0, sem_g)

        @pl.loop(0, L, step=2)
        def _l_loop(lbase):
            for p in range(2):
                l = lbase + p
                ib = ins[p]
                ob = outs[p]

                @pl.when(l + 1 < L)
                def _():
                    pltpu.async_copy(tab.at[idx_v.at[l + 1]], ins[1 - p], sem_g)

                pltpu.make_async_copy(tab.at[idx_v.at[l]], ib, sem_g).wait()

                @pl.when(l >= 2)
                def _():
                    pltpu.make_async_copy(
                        ob, out.at[pl.ds(base, BPW), l - 2], sem_o
                    ).wait()

                pj = [pos_v[l, pl.ds(_LANES * j, _LANES)] for j in range(nvec)]

                @pl.loop(0, BPW, step=4)
                def _r_loop(rbase):
                    for k in range(4):
                        r = rbase + k
                        for j in range(nvec):
                            sl = pl.ds(_LANES * j, _LANES)
                            ob[r, sl] = ib[r, sl] * scale + pj[j]

                pltpu.async_copy(ob, out.at[pl.ds(base, BPW), l], sem_o)

        pltpu.make_async_copy(o0, out.at[pl.ds(base, BPW), L - 2], sem_o).wait()
        pltpu.make_async_copy(o1, out.at[pl.ds(base, BPW), L - 1], sem_o).wait()

    kern = pl.kernel(
        body,
        out_type=jax.ShapeDtypeStruct((NW * BPW, L, D), jnp.float32),
        mesh=mesh,
        scratch_types=[
            pltpu.VMEM((L, BPW), jnp.int32),
            pltpu.VMEM((L, D), jnp.float32),
            pltpu.VMEM((BPW, D), jnp.float32),
            pltpu.VMEM((BPW, D), jnp.float32),
            pltpu.VMEM((BPW, D), jnp.float32),
            pltpu.VMEM((BPW, D), jnp.float32),
            pltpu.SemaphoreType.DMA,
            pltpu.SemaphoreType.DMA,
        ],
        compiler_params=pltpu.CompilerParams(use_tc_tiling_on_sc=False),
        interpret=interpret,
    )
    return kern(x_blk, table, pos_enc)


def kernel(x, table, pos_enc):
    B, L = x.shape
    BPW = B // _NW
    x_blk = x.reshape(_NW, BPW, L).transpose(0, 2, 1)
    return _run(x_blk, table, pos_enc)
